# baseline (speedup 1.0000x reference)
import jax
import jax.numpy as jnp
from jax import lax
from jax.experimental import pallas as pl
from jax.experimental.pallas import tpu as pltpu
from jax.experimental.pallas import tpu_sc as plsc

_B = 4096
_F = 26
_V = 100000
_D = 16
_EPS = 1e-3

_NC = 2
_NS = 16
_NW = _NC * _NS
_BPW = _B // _NW
_L = 16


def _sc_body(tbl_hbm, idx_hbm, bi_hbm, idx_v, vt_v, rows_v, sT_v, sqT_v,
             bi_v, sem):
    wid = lax.axis_index("s") * _NC + lax.axis_index("c")
    base_b = wid * _BPW

    pltpu.sync_copy(idx_hbm.at[:, pl.ds(base_b, _BPW)], idx_v)

    def vt_body(j, carry):
        f = j // (_BPW // _L)
        g = j % (_BPW // _L)
        o = pl.multiple_of(g * _L, _L)
        vt_v[f, pl.ds(o, _L)] = lax.shift_right_logical(
            idx_v[f, pl.ds(o, _L)], 3)
        return carry

    lax.fori_loop(0, _F * (_BPW // _L), vt_body, 0)

    def z_body(j, carry):
        r = j // (_BPW // _L)
        g = j % (_BPW // _L)
        o = pl.multiple_of(g * _L, _L)
        zero = jnp.zeros((_L,), jnp.float32)
        sT_v[r, pl.ds(o, _L)] = zero
        sqT_v[r, pl.ds(o, _L)] = zero
        return carry

    lax.fori_loop(0, _D * (_BPW // _L), z_body, 0)

    def fire(f, slot):
        return pltpu.async_copy(
            tbl_hbm.at[f].at[vt_v.at[f]],
            rows_v.at[slot],
            sem,
        )

    lanes = lax.broadcasted_iota(jnp.int32, (_L,), 0)

    def extract(f, slot):
        def g_body(g, carry):
            o = pl.multiple_of(g * _L, _L)
            vj = idx_v[f, pl.ds(o, _L)]
            sub = (vj & 7) * _D
            jvec = g * _L + lanes
            for d in range(_D):
                ed = plsc.load_gather(rows_v.at[slot], [jvec, sub + d])
                sT_v[d, pl.ds(o, _L)] = sT_v[d, pl.ds(o, _L)] + ed
                sqT_v[d, pl.ds(o, _L)] = sqT_v[d, pl.ds(o, _L)] + ed * ed
            return carry

        lax.fori_loop(0, _BPW // _L, g_body, 0)

    descs = [fire(0, 0)]
    for f in range(_F):
        if f + 1 < _F:
            descs.append(fire(f + 1, (f + 1) % 2))
        descs[f].wait()
        extract(f, f % 2)

    rows16 = lax.broadcasted_iota(jnp.int32, (_L,), 0)

    def bi_body(b, carry):
        col = jnp.full((_L,), b, jnp.int32)
        s = plsc.load_gather(sT_v, [rows16, col])
        sq = plsc.load_gather(sqT_v, [rows16, col])
        bi_v[b, :] = 0.5 * (s * s - sq)
        return carry

    lax.fori_loop(0, _BPW, bi_body, 0)

    pltpu.sync_copy(bi_v, bi_hbm.at[pl.ds(base_b, _BPW)])


@jax.jit
def _sc_gather_pool(tbl, idxT):
    mesh = plsc.VectorSubcoreMesh(core_axis_name="c", subcore_axis_name="s")
    return pl.kernel(
        _sc_body,
        out_type=jax.ShapeDtypeStruct((_B, _D), jnp.float32),
        mesh=mesh,
        scratch_types=[
            pltpu.VMEM((_F, _BPW), jnp.int32),
            pltpu.VMEM((_F, _BPW), jnp.int32),
            pltpu.VMEM((2, _BPW, 128), jnp.float32),
            pltpu.VMEM((_D, _BPW), jnp.float32),
            pltpu.VMEM((_D, _BPW), jnp.float32),
            pltpu.VMEM((_BPW, _D), jnp.float32),
            pltpu.SemaphoreType.DMA,
        ],
        compiler_params=pltpu.CompilerParams(use_tc_tiling_on_sc=True,
                                             needs_layout_passes=False),
    )(tbl, idxT)


def _tc_body(bi_ref, gamma_ref, beta_ref, W1_ref, b1_ref, W2_ref, b2_ref,
             W3_ref, b3_ref, Wo_ref, bo_ref, out_ref):
    bi = bi_ref[...]
    mean = jnp.mean(bi, axis=0, keepdims=True)
    var = jnp.mean((bi - mean) ** 2, axis=0, keepdims=True)
    x = (bi - mean) * lax.rsqrt(var + _EPS) * gamma_ref[...] + beta_ref[...]
    x = jnp.maximum(jnp.dot(x, W1_ref[...],
                            preferred_element_type=jnp.float32)
                    + b1_ref[...], 0.0)
    x = jnp.maximum(jnp.dot(x, W2_ref[...],
                            preferred_element_type=jnp.float32)
                    + b2_ref[...], 0.0)
    x = jnp.maximum(jnp.dot(x, W3_ref[...],
                            preferred_element_type=jnp.float32)
                    + b3_ref[...], 0.0)
    z = jnp.dot(x, Wo_ref[...], preferred_element_type=jnp.float32) \
        + bo_ref[...]
    out_ref[...] = 1.0 / (1.0 + jnp.exp(-z))


@jax.jit
def _tc_bn_mlp(bi, gamma, beta, W1, b1, W2, b2, W3, b3, Wo, bo):
    return pl.pallas_call(
        _tc_body,
        out_shape=jax.ShapeDtypeStruct((_B, 1), jnp.float32),
    )(bi, gamma, beta, W1, b1, W2, b2, W3, b3, Wo, bo)


def kernel(inputs, tables, gamma, beta, W1, b1, W2, b2, W3, b3, Wo, bo):
    tbl = tables.reshape(_F, _V // 8, 8 * _D)
    idxT = inputs.T
    bi = _sc_gather_pool(tbl, idxT)
    return _tc_bn_mlp(
        bi, gamma.reshape(1, _D), beta.reshape(1, _D),
        W1, b1.reshape(1, -1), W2, b2.reshape(1, -1),
        W3, b3.reshape(1, -1), Wo, bo.reshape(1, 1),
    )
